# baseline (speedup 1.0000x reference)
import functools

import jax
import jax.numpy as jnp
from jax import lax
from jax.experimental import pallas as pl
from jax.experimental.pallas import tpu as pltpu
from jax.experimental.pallas import tpu_sc as plsc


def _rsqrt16(s):
    s = jnp.maximum(s, jnp.float32(1e-24))
    i = plsc.bitcast(s, jnp.int32)
    i = jnp.int32(0x5F3759DF) - (i >> 1)
    y = plsc.bitcast(i, jnp.float32)
    for _ in range(3):
        y = y * (jnp.float32(1.5) - jnp.float32(0.5) * s * y * y)
    return y


def _make_sc_kernel(n_rel, dim, B, NEG, margin):
    info = plsc.get_sparse_core_info()
    NC, NS = info.num_cores, info.num_subcores
    NW = NC * NS
    P = B // NW
    CH = NEG // (NW * P)
    SLABS = P // 128
    assert dim == 32 and P % 128 == 0 and NEG == NW * P * CH and B == NW * P

    mesh = plsc.VectorSubcoreMesh(core_axis_name="c", subcore_axis_name="s")

    def chunk_scores(rows_h_v, rows_t_v, rel_v, idx_r_v, it, lanes):
        rvec = lanes + it * 16
        ridx = idx_r_v[it >> 3, pl.ds((it & 7) * 16, 16)]
        sh = jnp.zeros((16,), jnp.float32)
        st = jnp.zeros((16,), jnp.float32)
        for d in range(dim):
            dv = (lanes + d) & (dim - 1)
            hv = plsc.load_gather(rows_h_v, [rvec, dv])
            tv = plsc.load_gather(rows_t_v, [rvec, dv])
            sh = sh + hv * hv
            st = st + tv * tv
        ih = _rsqrt16(sh)
        iv = _rsqrt16(st)
        s = jnp.zeros((16,), jnp.float32)
        for d in range(dim):
            dv = (lanes + d) & (dim - 1)
            hv = plsc.load_gather(rows_h_v, [rvec, dv])
            tv = plsc.load_gather(rows_t_v, [rvec, dv])
            rv = plsc.load_gather(rel_v, [ridx, dv])
            s = s + jnp.abs(hv * ih + rv - tv * iv)
        return -s

    def load_chunk(src_h, src_t, src_r, slab0, idx_h_v, idx_t_v, idx_r_v,
                   rows_h_v, rows_t_v, ent, sem):
        pltpu.sync_copy(src_h.at[pl.ds(slab0, SLABS)], idx_h_v)
        pltpu.sync_copy(src_t.at[pl.ds(slab0, SLABS)], idx_t_v)
        pltpu.sync_copy(src_r.at[pl.ds(slab0, SLABS)], idx_r_v)
        cps = []
        for j in range(SLABS):
            cps.append(pltpu.async_copy(
                ent.at[idx_h_v.at[j]], rows_h_v.at[pl.ds(j * 128, 128)], sem))
            cps.append(pltpu.async_copy(
                ent.at[idx_t_v.at[j]], rows_t_v.at[pl.ds(j * 128, 128)], sem))
        for c in cps:
            c.wait()

    @functools.partial(
        pl.kernel, mesh=mesh,
        out_type=jax.ShapeDtypeStruct((NW, 16), jnp.float32),
        compiler_params=pltpu.CompilerParams(
            needs_layout_passes=False, use_tc_tiling_on_sc=False),
        scratch_types=[
            pltpu.VMEM((n_rel, dim), jnp.float32),
            pltpu.VMEM((SLABS, 128), jnp.int32),
            pltpu.VMEM((SLABS, 128), jnp.int32),
            pltpu.VMEM((SLABS, 128), jnp.int32),
            pltpu.VMEM((P, 32), jnp.float32),
            pltpu.VMEM((P, 32), jnp.float32),
            pltpu.VMEM((P,), jnp.float32),
            pltpu.VMEM((16,), jnp.float32),
            pltpu.SemaphoreType.DMA,
        ])
    def sc_body(ph, pr, pt, nh, nr, nt, ent, rel, out,
                rel_v, idx_h_v, idx_t_v, idx_r_v, rows_h_v, rows_t_v,
                pos_s_v, acc_v, sem):
        wid = lax.axis_index("s") * NC + lax.axis_index("c")
        lanes = lax.iota(jnp.int32, 16)
        pltpu.sync_copy(rel, rel_v)

        load_chunk(ph, pt, pr, wid * SLABS, idx_h_v, idx_t_v, idx_r_v,
                   rows_h_v, rows_t_v, ent, sem)

        def pos_it(it, carry):
            sc = chunk_scores(rows_h_v, rows_t_v, rel_v, idx_r_v, it, lanes)
            pos_s_v[pl.ds(it * 16, 16)] = sc
            return carry
        lax.fori_loop(0, P // 16, pos_it, jnp.int32(0))

        iota4 = lanes >> 2
        acc = jnp.zeros((16,), jnp.float32)
        for n in range(CH):
            load_chunk(nh, nt, nr, (wid * CH + n) * SLABS,
                       idx_h_v, idx_t_v, idx_r_v, rows_h_v, rows_t_v, ent, sem)

            def neg_it(it, a, _n=n):
                sc = chunk_scores(rows_h_v, rows_t_v, rel_v, idx_r_v, it, lanes)
                pidx = iota4 + (_n * 128 + it * 4)
                ps = plsc.load_gather(pos_s_v, [pidx])
                return a + jnp.maximum(jnp.float32(margin) - ps + sc,
                                       jnp.float32(0.0))
            acc = lax.fori_loop(0, P // 16, neg_it, acc)

        acc_v[...] = acc
        pltpu.sync_copy(acc_v, out.at[wid])

    return sc_body


def _reduce_loss(partials, neg_total):
    def body(p_ref, o_ref):
        o_ref[0, 0] = jnp.sum(p_ref[...]) * jnp.float32(1.0 / neg_total)

    out = pl.pallas_call(
        body,
        out_shape=jax.ShapeDtypeStruct((1, 1), jnp.float32),
        out_specs=pl.BlockSpec(memory_space=pltpu.SMEM),
    )(partials)
    return out[0, 0]


def kernel(pos_h, pos_r, pos_t, neg_h, neg_r, neg_t, ent_emb, rel_emb):
    B = pos_h.shape[0]
    NEG = neg_h.shape[0]
    n_rel, dim = rel_emb.shape
    sck = _make_sc_kernel(n_rel, dim, B, NEG, margin=1.0)
    ph = pos_h.reshape(-1, 128)
    pr = pos_r.reshape(-1, 128)
    pt = pos_t.reshape(-1, 128)
    nh = neg_h.reshape(-1, 128)
    nr = neg_r.reshape(-1, 128)
    nt = neg_t.reshape(-1, 128)
    partials = sck(ph, pr, pt, nh, nr, nt, ent_emb, rel_emb)
    return _reduce_loss(partials, NEG)


# speedup vs baseline: 1.7383x; 1.7383x over previous
import functools

import jax
import jax.numpy as jnp
from jax import lax
from jax.experimental import pallas as pl
from jax.experimental.pallas import tpu as pltpu
from jax.experimental.pallas import tpu_sc as plsc


def _rsqrt16(s):
    s = jnp.maximum(s, jnp.float32(1e-24))
    i = plsc.bitcast(s, jnp.int32)
    i = jnp.int32(0x5F3759DF) - (i >> 1)
    y = plsc.bitcast(i, jnp.float32)
    for _ in range(3):
        y = y * (jnp.float32(1.5) - jnp.float32(0.5) * s * y * y)
    return y


def _make_sc_kernel(n_rel, dim, B, NEG, margin, ent_blk):
    info = plsc.get_sparse_core_info()
    NC, NS = info.num_cores, info.num_subcores
    NW = NC * NS
    P = B // NW
    CH = NEG // (NW * P)
    SLABS = P // 128
    assert dim == 32 and P % 128 == 0 and NEG == NW * P * CH and B == NW * P

    mesh = plsc.VectorSubcoreMesh(core_axis_name="c", subcore_axis_name="s")

    def chunk_scores(rows_h_v, rows_t_v, rel_v, idx_r_v, it, lanes):
        rvec = lanes + it * 16
        ridx = idx_r_v[it >> 3, pl.ds((it & 7) * 16, 16)]
        sh = jnp.zeros((16,), jnp.float32)
        st = jnp.zeros((16,), jnp.float32)
        for d in range(dim):
            dv = (lanes + d) & (dim - 1)
            hv = plsc.load_gather(rows_h_v, [rvec, dv])
            tv = plsc.load_gather(rows_t_v, [rvec, dv])
            sh = sh + hv * hv
            st = st + tv * tv
        ih = _rsqrt16(sh)
        iv = _rsqrt16(st)
        s = jnp.zeros((16,), jnp.float32)
        for d in range(dim):
            dv = (lanes + d) & (dim - 1)
            hv = plsc.load_gather(rows_h_v, [rvec, dv])
            tv = plsc.load_gather(rows_t_v, [rvec, dv])
            rv = plsc.load_gather(rel_v, [ridx, dv])
            s = s + jnp.abs(hv * ih + rv - tv * iv)
        return -s

    ql = ent_blk // 4
    qsh = ql.bit_length() - 1
    assert ent_blk & (ent_blk - 1) == 0 and (1 << qsh) == ql

    def xform_ent(idx_ref):
        def body(i, c):
            v = idx_ref[i >> 3, pl.ds((i & 7) * 16, 16)]
            g = (v & (-ent_blk)) + ((v & (ql - 1)) << 2) + ((v >> qsh) & 3)
            idx_ref[i >> 3, pl.ds((i & 7) * 16, 16)] = g
            return c
        lax.fori_loop(0, SLABS * 8, body, jnp.int32(0))

    def load_chunk(src_h, src_t, src_r, slab0, idx_h_v, idx_t_v, idx_r_v,
                   rows_h_v, rows_t_v, ent, sem):
        pltpu.sync_copy(src_h.at[pl.ds(slab0, SLABS)], idx_h_v)
        pltpu.sync_copy(src_t.at[pl.ds(slab0, SLABS)], idx_t_v)
        pltpu.sync_copy(src_r.at[pl.ds(slab0, SLABS)], idx_r_v)
        xform_ent(idx_h_v)
        xform_ent(idx_t_v)
        cps = []
        for j in range(SLABS):
            cps.append(pltpu.async_copy(
                ent.at[idx_h_v.at[j]], rows_h_v.at[pl.ds(j * 128, 128)], sem))
            cps.append(pltpu.async_copy(
                ent.at[idx_t_v.at[j]], rows_t_v.at[pl.ds(j * 128, 128)], sem))
        for c in cps:
            c.wait()

    @functools.partial(
        pl.kernel, mesh=mesh,
        out_type=jax.ShapeDtypeStruct((NW, 16), jnp.float32),
        compiler_params=pltpu.CompilerParams(
            needs_layout_passes=False, use_tc_tiling_on_sc=False),
        scratch_types=[
            pltpu.VMEM((n_rel, dim), jnp.float32),
            pltpu.VMEM((SLABS, 128), jnp.int32),
            pltpu.VMEM((SLABS, 128), jnp.int32),
            pltpu.VMEM((SLABS, 128), jnp.int32),
            pltpu.VMEM((P, 32), jnp.float32),
            pltpu.VMEM((P, 32), jnp.float32),
            pltpu.VMEM((P,), jnp.float32),
            pltpu.VMEM((16,), jnp.float32),
            pltpu.SemaphoreType.DMA,
        ])
    def sc_body(ph, pr, pt, nh, nr, nt, ent, rel, out,
                rel_v, idx_h_v, idx_t_v, idx_r_v, rows_h_v, rows_t_v,
                pos_s_v, acc_v, sem):
        wid = lax.axis_index("s") * NC + lax.axis_index("c")
        lanes = lax.iota(jnp.int32, 16)
        pltpu.sync_copy(rel, rel_v)

        load_chunk(ph, pt, pr, wid * SLABS, idx_h_v, idx_t_v, idx_r_v,
                   rows_h_v, rows_t_v, ent, sem)

        def pos_it(it, carry):
            sc = chunk_scores(rows_h_v, rows_t_v, rel_v, idx_r_v, it, lanes)
            pos_s_v[pl.ds(it * 16, 16)] = sc
            return carry
        lax.fori_loop(0, P // 16, pos_it, jnp.int32(0))

        iota4 = lanes >> 2
        acc = jnp.zeros((16,), jnp.float32)
        for n in range(CH):
            load_chunk(nh, nt, nr, (wid * CH + n) * SLABS,
                       idx_h_v, idx_t_v, idx_r_v, rows_h_v, rows_t_v, ent, sem)

            def neg_it(it, a, _n=n):
                sc = chunk_scores(rows_h_v, rows_t_v, rel_v, idx_r_v, it, lanes)
                pidx = iota4 + (_n * 128 + it * 4)
                ps = plsc.load_gather(pos_s_v, [pidx])
                return a + jnp.maximum(jnp.float32(margin) - ps + sc,
                                       jnp.float32(0.0))
            acc = lax.fori_loop(0, P // 16, neg_it, acc)

        acc_v[...] = acc
        pltpu.sync_copy(acc_v, out.at[wid])

    return sc_body


def _compact_body(x_ref, o_ref):
    dim, blk = x_ref.shape
    L = blk // 4
    eye = jnp.eye(dim, dtype=jnp.float32)
    dn = (((0,), (0,)), ((), ()))
    parts = [
        jax.lax.dot_general(x_ref[:, a * L:(a + 1) * L], eye, dn,
                            preferred_element_type=jnp.float32)
        for a in range(4)
    ]
    o_ref[...] = jnp.concatenate(parts, axis=1)


def _compact_table(x, blk):
    n, dim = x.shape
    x_t = x.T
    grid = pl.cdiv(n, blk)
    out = pl.pallas_call(
        _compact_body,
        grid=(grid,),
        in_specs=[pl.BlockSpec((dim, blk), lambda i: (0, i))],
        out_specs=pl.BlockSpec((blk // 4, 4 * dim), lambda i: (i, 0)),
        out_shape=jax.ShapeDtypeStruct((grid * blk // 4, 4 * dim), jnp.float32),
    )(x_t)
    return out.reshape(grid * blk, dim)


def _reduce_loss(partials, neg_total):
    def body(p_ref, o_ref):
        o_ref[0, 0] = jnp.sum(p_ref[...]) * jnp.float32(1.0 / neg_total)

    out = pl.pallas_call(
        body,
        out_shape=jax.ShapeDtypeStruct((1, 1), jnp.float32),
        out_specs=pl.BlockSpec(memory_space=pltpu.SMEM),
    )(partials)
    return out[0, 0]


def kernel(pos_h, pos_r, pos_t, neg_h, neg_r, neg_t, ent_emb, rel_emb):
    B = pos_h.shape[0]
    NEG = neg_h.shape[0]
    n_rel, dim = rel_emb.shape
    sck = _make_sc_kernel(n_rel, dim, B, NEG, margin=1.0, ent_blk=8192)
    ph = pos_h.reshape(-1, 128)
    pr = pos_r.reshape(-1, 128)
    pt = pos_t.reshape(-1, 128)
    nh = neg_h.reshape(-1, 128)
    nr = neg_r.reshape(-1, 128)
    nt = neg_t.reshape(-1, 128)
    ent_c = _compact_table(ent_emb, blk=8192)
    partials = sck(ph, pr, pt, nh, nr, nt, ent_c, rel_emb)
    return _reduce_loss(partials, NEG)


# speedup vs baseline: 2.6524x; 1.5258x over previous
import functools

import jax
import jax.numpy as jnp
from jax import lax
from jax.experimental import pallas as pl
from jax.experimental.pallas import tpu as pltpu
from jax.experimental.pallas import tpu_sc as plsc


def _rsqrt16(s):
    s = jnp.maximum(s, jnp.float32(1e-24))
    i = plsc.bitcast(s, jnp.int32)
    i = jnp.int32(0x5F3759DF) - (i >> 1)
    y = plsc.bitcast(i, jnp.float32)
    for _ in range(3):
        y = y * (jnp.float32(1.5) - jnp.float32(0.5) * s * y * y)
    return y


def _make_sc_kernel(n_rel, dim, B, NEG, margin, ent_blk):
    info = plsc.get_sparse_core_info()
    NC, NS = info.num_cores, info.num_subcores
    NW = NC * NS
    P = B // NW
    CH = NEG // (NW * P)
    SLABS = P // 128
    assert dim == 32 and P % 128 == 0 and NEG == NW * P * CH and B == NW * P

    mesh = plsc.VectorSubcoreMesh(core_axis_name="c", subcore_axis_name="s")

    def chunk_scores(rows_h_v, rows_t_v, rel_v, idx_r_v, it, lanes):
        rvec = lanes + it * 16
        ridx = idx_r_v[it >> 3, pl.ds((it & 7) * 16, 16)]
        sh = jnp.zeros((16,), jnp.float32)
        st = jnp.zeros((16,), jnp.float32)
        for d in range(dim):
            dv = (lanes + d) & (dim - 1)
            hv = plsc.load_gather(rows_h_v, [rvec, dv])
            tv = plsc.load_gather(rows_t_v, [rvec, dv])
            sh = sh + hv * hv
            st = st + tv * tv
        ih = _rsqrt16(sh)
        iv = _rsqrt16(st)
        s = jnp.zeros((16,), jnp.float32)
        for d in range(dim):
            dv = (lanes + d) & (dim - 1)
            hv = plsc.load_gather(rows_h_v, [rvec, dv])
            tv = plsc.load_gather(rows_t_v, [rvec, dv])
            rv = plsc.load_gather(rel_v, [ridx, dv])
            s = s + jnp.abs(hv * ih + rv - tv * iv)
        return -s

    ql = ent_blk // 4
    qsh = ql.bit_length() - 1
    assert ent_blk & (ent_blk - 1) == 0 and (1 << qsh) == ql

    def xform_ent(idx_ref):
        def body(i, c):
            v = idx_ref[i >> 3, pl.ds((i & 7) * 16, 16)]
            g = (v & (-ent_blk)) + ((v & (ql - 1)) << 2) + ((v >> qsh) & 3)
            idx_ref[i >> 3, pl.ds((i & 7) * 16, 16)] = g
            return c
        lax.fori_loop(0, SLABS * 8, body, jnp.int32(0))

    def load_chunk(src_h, src_t, src_r, slab0, idx_h_v, idx_t_v, idx_r_v,
                   rows_h_v, rows_t_v, ent, sem):
        pltpu.sync_copy(src_h.at[pl.ds(slab0, SLABS)], idx_h_v)
        pltpu.sync_copy(src_t.at[pl.ds(slab0, SLABS)], idx_t_v)
        pltpu.sync_copy(src_r.at[pl.ds(slab0, SLABS)], idx_r_v)
        xform_ent(idx_h_v)
        xform_ent(idx_t_v)
        cps = []
        for j in range(SLABS):
            cps.append(pltpu.async_copy(
                ent.at[idx_h_v.at[j]], rows_h_v.at[pl.ds(j * 128, 128)], sem))
            cps.append(pltpu.async_copy(
                ent.at[idx_t_v.at[j]], rows_t_v.at[pl.ds(j * 128, 128)], sem))
        for c in cps:
            c.wait()

    @functools.partial(
        pl.kernel, mesh=mesh,
        out_type=jax.ShapeDtypeStruct((NW, 16), jnp.float32),
        compiler_params=pltpu.CompilerParams(
            needs_layout_passes=False, use_tc_tiling_on_sc=False),
        scratch_types=[
            pltpu.VMEM((n_rel, dim), jnp.float32),
            pltpu.VMEM((SLABS, 128), jnp.int32),
            pltpu.VMEM((SLABS, 128), jnp.int32),
            pltpu.VMEM((SLABS, 128), jnp.int32),
            pltpu.VMEM((P, 32), jnp.float32),
            pltpu.VMEM((P, 32), jnp.float32),
            pltpu.VMEM((P,), jnp.float32),
            pltpu.VMEM((16,), jnp.float32),
            pltpu.SemaphoreType.DMA,
        ])
    def sc_body(ph, pr, pt, nh, nr, nt, ent, rel, out,
                rel_v, idx_h_v, idx_t_v, idx_r_v, rows_h_v, rows_t_v,
                pos_s_v, acc_v, sem):
        wid = lax.axis_index("s") * NC + lax.axis_index("c")
        lanes = lax.iota(jnp.int32, 16)
        pltpu.sync_copy(rel, rel_v)

        load_chunk(ph, pt, pr, wid * SLABS, idx_h_v, idx_t_v, idx_r_v,
                   rows_h_v, rows_t_v, ent, sem)

        def pos_it(it, carry):
            sc = chunk_scores(rows_h_v, rows_t_v, rel_v, idx_r_v, it, lanes)
            pos_s_v[pl.ds(it * 16, 16)] = sc
            return carry
        lax.fori_loop(0, P // 16, pos_it, jnp.int32(0))

        iota4 = lanes >> 2
        acc = jnp.zeros((16,), jnp.float32)
        for n in range(CH):
            load_chunk(nh, nt, nr, (wid * CH + n) * SLABS,
                       idx_h_v, idx_t_v, idx_r_v, rows_h_v, rows_t_v, ent, sem)

            def neg_it(it, a, _n=n):
                sc = chunk_scores(rows_h_v, rows_t_v, rel_v, idx_r_v, it, lanes)
                pidx = iota4 + (_n * 128 + it * 4)
                ps = plsc.load_gather(pos_s_v, [pidx])
                return a + jnp.maximum(jnp.float32(margin) - ps + sc,
                                       jnp.float32(0.0))
            acc = lax.fori_loop(0, P // 16, neg_it, acc)

        acc_v[...] = acc
        pltpu.sync_copy(acc_v, out.at[wid])

    return sc_body


def _compact_body(x_ref, o_ref):
    dim, blk = x_ref.shape
    L = blk // 4
    stacked = jnp.concatenate(
        [x_ref[:, a * L:(a + 1) * L] for a in range(4)], axis=0)
    eye = jnp.eye(4 * dim, dtype=jnp.float32)
    dn = (((0,), (0,)), ((), ()))
    o_ref[...] = jax.lax.dot_general(stacked, eye, dn,
                                     preferred_element_type=jnp.float32)


def _compact_table(x, blk):
    n, dim = x.shape
    x_t = x.T
    grid = pl.cdiv(n, blk)
    out = pl.pallas_call(
        _compact_body,
        grid=(grid,),
        in_specs=[pl.BlockSpec((dim, blk), lambda i: (0, i))],
        out_specs=pl.BlockSpec((blk // 4, 4 * dim), lambda i: (i, 0)),
        out_shape=jax.ShapeDtypeStruct((grid * blk // 4, 4 * dim), jnp.float32),
    )(x_t)
    return out.reshape(grid * blk, dim)


def _reduce_loss(partials, neg_total):
    def body(p_ref, o_ref):
        o_ref[0, 0] = jnp.sum(p_ref[...]) * jnp.float32(1.0 / neg_total)

    out = pl.pallas_call(
        body,
        out_shape=jax.ShapeDtypeStruct((1, 1), jnp.float32),
        out_specs=pl.BlockSpec(memory_space=pltpu.SMEM),
    )(partials)
    return out[0, 0]


def kernel(pos_h, pos_r, pos_t, neg_h, neg_r, neg_t, ent_emb, rel_emb):
    B = pos_h.shape[0]
    NEG = neg_h.shape[0]
    n_rel, dim = rel_emb.shape
    sck = _make_sc_kernel(n_rel, dim, B, NEG, margin=1.0, ent_blk=8192)
    ph = pos_h.reshape(-1, 128)
    pr = pos_r.reshape(-1, 128)
    pt = pos_t.reshape(-1, 128)
    nh = neg_h.reshape(-1, 128)
    nr = neg_r.reshape(-1, 128)
    nt = neg_t.reshape(-1, 128)
    ent_c = _compact_table(ent_emb, blk=8192)
    partials = sck(ph, pr, pt, nh, nr, nt, ent_c, rel_emb)
    return _reduce_loss(partials, NEG)


# speedup vs baseline: 3.5634x; 1.3435x over previous
import functools

import jax
import jax.numpy as jnp
from jax import lax
from jax.experimental import pallas as pl
from jax.experimental.pallas import tpu as pltpu
from jax.experimental.pallas import tpu_sc as plsc


def _rsqrt16(s):
    s = jnp.maximum(s, jnp.float32(1e-24))
    i = plsc.bitcast(s, jnp.int32)
    i = jnp.int32(0x5F3759DF) - (i >> 1)
    y = plsc.bitcast(i, jnp.float32)
    for _ in range(3):
        y = y * (jnp.float32(1.5) - jnp.float32(0.5) * s * y * y)
    return y


def _make_sc_kernel(n_rel, dim, B, NEG, margin, ent_blk):
    info = plsc.get_sparse_core_info()
    NC, NS = info.num_cores, info.num_subcores
    NW = NC * NS
    P = B // NW
    CH = NEG // (NW * P)
    SLABS = P // 128
    assert dim == 32 and P % 128 == 0 and NEG == NW * P * CH and B == NW * P

    mesh = plsc.VectorSubcoreMesh(core_axis_name="c", subcore_axis_name="s")

    def chunk_scores(rows_h_v, rows_t_v, rel_v, idx_r_v, it, lanes):
        rvec = lanes + it * 16
        ridx = idx_r_v[it >> 3, pl.ds((it & 7) * 16, 16)]
        sh = jnp.zeros((16,), jnp.float32)
        st = jnp.zeros((16,), jnp.float32)
        for d in range(dim):
            dv = (lanes + d) & (dim - 1)
            hv = plsc.load_gather(rows_h_v, [rvec, dv])
            tv = plsc.load_gather(rows_t_v, [rvec, dv])
            sh = sh + hv * hv
            st = st + tv * tv
        ih = _rsqrt16(sh)
        iv = _rsqrt16(st)
        s = jnp.zeros((16,), jnp.float32)
        for d in range(dim):
            dv = (lanes + d) & (dim - 1)
            hv = plsc.load_gather(rows_h_v, [rvec, dv])
            tv = plsc.load_gather(rows_t_v, [rvec, dv])
            rv = plsc.load_gather(rel_v, [ridx, dv])
            s = s + jnp.abs(hv * ih + rv - tv * iv)
        return -s

    ql = ent_blk // 4
    qsh = ql.bit_length() - 1
    assert ent_blk & (ent_blk - 1) == 0 and (1 << qsh) == ql

    def xform_ent(idx_ref):
        def body(i, c):
            v = idx_ref[i >> 3, pl.ds((i & 7) * 16, 16)]
            g = (v & (-ent_blk)) + ((v & (ql - 1)) << 2) + ((v >> qsh) & 3)
            idx_ref[i >> 3, pl.ds((i & 7) * 16, 16)] = g
            return c
        lax.fori_loop(0, SLABS * 8, body, jnp.int32(0))

    def load_chunk(src_h, src_t, src_r, slab0, idx_h_v, idx_t_v, idx_r_v,
                   rows_h_v, rows_t_v, ent, sem):
        pltpu.sync_copy(src_h.at[pl.ds(slab0, SLABS)], idx_h_v)
        pltpu.sync_copy(src_t.at[pl.ds(slab0, SLABS)], idx_t_v)
        pltpu.sync_copy(src_r.at[pl.ds(slab0, SLABS)], idx_r_v)
        xform_ent(idx_h_v)
        xform_ent(idx_t_v)
        cps = []
        for j in range(SLABS):
            cps.append(pltpu.async_copy(
                ent.at[idx_h_v.at[j]], rows_h_v.at[pl.ds(j * 128, 128)], sem))
            cps.append(pltpu.async_copy(
                ent.at[idx_t_v.at[j]], rows_t_v.at[pl.ds(j * 128, 128)], sem))
        for c in cps:
            c.wait()

    @functools.partial(
        pl.kernel, mesh=mesh,
        out_type=jax.ShapeDtypeStruct((NW, 16), jnp.float32),
        compiler_params=pltpu.CompilerParams(
            needs_layout_passes=False, use_tc_tiling_on_sc=False),
        scratch_types=[
            pltpu.VMEM((n_rel, dim), jnp.float32),
            pltpu.VMEM((SLABS, 128), jnp.int32),
            pltpu.VMEM((SLABS, 128), jnp.int32),
            pltpu.VMEM((SLABS, 128), jnp.int32),
            pltpu.VMEM((P, 32), jnp.float32),
            pltpu.VMEM((P, 32), jnp.float32),
            pltpu.VMEM((P,), jnp.float32),
            pltpu.VMEM((16,), jnp.float32),
            pltpu.SemaphoreType.DMA,
        ])
    def sc_body(ph, pr, pt, nh, nr, nt, ent, rel, out,
                rel_v, idx_h_v, idx_t_v, idx_r_v, rows_h_v, rows_t_v,
                pos_s_v, acc_v, sem):
        wid = lax.axis_index("s") * NC + lax.axis_index("c")
        lanes = lax.iota(jnp.int32, 16)
        pltpu.sync_copy(rel, rel_v)

        load_chunk(ph, pt, pr, wid * SLABS, idx_h_v, idx_t_v, idx_r_v,
                   rows_h_v, rows_t_v, ent, sem)

        def pos_it(it, carry):
            sc = chunk_scores(rows_h_v, rows_t_v, rel_v, idx_r_v, it, lanes)
            pos_s_v[pl.ds(it * 16, 16)] = sc
            return carry
        lax.fori_loop(0, P // 16, pos_it, jnp.int32(0))

        iota4 = lanes >> 2
        acc = jnp.zeros((16,), jnp.float32)
        for n in range(CH):
            load_chunk(nh, nt, nr, (wid * CH + n) * SLABS,
                       idx_h_v, idx_t_v, idx_r_v, rows_h_v, rows_t_v, ent, sem)

            def neg_it(it, a, _n=n):
                sc = chunk_scores(rows_h_v, rows_t_v, rel_v, idx_r_v, it, lanes)
                pidx = iota4 + (_n * 128 + it * 4)
                ps = plsc.load_gather(pos_s_v, [pidx])
                return a + jnp.maximum(jnp.float32(margin) - ps + sc,
                                       jnp.float32(0.0))
            acc = lax.fori_loop(0, P // 16, neg_it, acc)

        acc_v[...] = acc
        pltpu.sync_copy(acc_v, out.at[wid])

    return sc_body


def _compact_body(x_ref, o_ref):
    dim, blk = x_ref.shape
    L = blk // 4
    stacked = jnp.concatenate(
        [x_ref[:, a * L:(a + 1) * L] for a in range(4)], axis=0)
    eye = jnp.eye(4 * dim, dtype=jnp.float32)
    dn = (((0,), (0,)), ((), ()))
    o_ref[...] = jax.lax.dot_general(stacked, eye, dn,
                                     preferred_element_type=jnp.float32)


def _compact_table(x, blk):
    n, dim = x.shape
    x_t = x.T
    grid = pl.cdiv(n, blk)
    out = pl.pallas_call(
        _compact_body,
        grid=(grid,),
        in_specs=[pl.BlockSpec((dim, blk), lambda i: (0, i))],
        out_specs=pl.BlockSpec((blk // 4, 4 * dim), lambda i: (i, 0)),
        out_shape=jax.ShapeDtypeStruct((grid * blk // 4, 4 * dim), jnp.float32),
    )(x_t)
    return out.reshape(grid * blk, dim)


def _reduce_loss(partials, neg_total):
    def body(p_ref, o_ref):
        o_ref[0, 0] = jnp.sum(p_ref[...]) * jnp.float32(1.0 / neg_total)

    out = pl.pallas_call(
        body,
        out_shape=jax.ShapeDtypeStruct((1, 1), jnp.float32),
        out_specs=pl.BlockSpec(memory_space=pltpu.SMEM),
    )(partials)
    return out[0, 0]


def kernel(pos_h, pos_r, pos_t, neg_h, neg_r, neg_t, ent_emb, rel_emb):
    B = pos_h.shape[0]
    NEG = neg_h.shape[0]
    n_rel, dim = rel_emb.shape
    sck = _make_sc_kernel(n_rel, dim, B, NEG, margin=1.0, ent_blk=32768)
    ph = pos_h.reshape(-1, 128)
    pr = pos_r.reshape(-1, 128)
    pt = pos_t.reshape(-1, 128)
    nh = neg_h.reshape(-1, 128)
    nr = neg_r.reshape(-1, 128)
    nt = neg_t.reshape(-1, 128)
    ent_c = _compact_table(ent_emb, blk=32768)
    partials = sck(ph, pr, pt, nh, nr, nt, ent_c, rel_emb)
    return _reduce_loss(partials, NEG)


# speedup vs baseline: 3.5841x; 1.0058x over previous
import functools

import jax
import jax.numpy as jnp
from jax import lax
from jax.experimental import pallas as pl
from jax.experimental.pallas import tpu as pltpu
from jax.experimental.pallas import tpu_sc as plsc


def _rsqrt16(s):
    s = jnp.maximum(s, jnp.float32(1e-24))
    i = plsc.bitcast(s, jnp.int32)
    i = jnp.int32(0x5F3759DF) - (i >> 1)
    y = plsc.bitcast(i, jnp.float32)
    for _ in range(3):
        y = y * (jnp.float32(1.5) - jnp.float32(0.5) * s * y * y)
    return y


def _make_sc_kernel(n_rel, dim, B, NEG, margin, ent_blk):
    info = plsc.get_sparse_core_info()
    NC, NS = info.num_cores, info.num_subcores
    NW = NC * NS
    P = B // NW
    CH = NEG // (NW * P)
    SLABS = P // 128
    assert dim == 32 and P % 128 == 0 and NEG == NW * P * CH and B == NW * P

    mesh = plsc.VectorSubcoreMesh(core_axis_name="c", subcore_axis_name="s")

    def chunk_scores(rows_h_v, rows_t_v, rel_v, idx_r_v, it, lanes):
        rvec = lanes + it * 16
        ridx = idx_r_v[it >> 3, pl.ds((it & 7) * 16, 16)]
        sh = jnp.zeros((16,), jnp.float32)
        st = jnp.zeros((16,), jnp.float32)
        for d in range(dim):
            dv = (lanes + d) & (dim - 1)
            hv = plsc.load_gather(rows_h_v, [rvec, dv])
            tv = plsc.load_gather(rows_t_v, [rvec, dv])
            sh = sh + hv * hv
            st = st + tv * tv
        ih = _rsqrt16(sh)
        iv = _rsqrt16(st)
        s = jnp.zeros((16,), jnp.float32)
        for d in range(dim):
            dv = (lanes + d) & (dim - 1)
            hv = plsc.load_gather(rows_h_v, [rvec, dv])
            tv = plsc.load_gather(rows_t_v, [rvec, dv])
            rv = plsc.load_gather(rel_v, [ridx, dv])
            s = s + jnp.abs(hv * ih + rv - tv * iv)
        return -s

    ql = ent_blk // 4
    qsh = ql.bit_length() - 1
    assert ent_blk & (ent_blk - 1) == 0 and (1 << qsh) == ql

    def xform_ent(idx_ref):
        def body(i, c):
            v = idx_ref[i >> 3, pl.ds((i & 7) * 16, 16)]
            g = (v & (-ent_blk)) + ((v & (ql - 1)) << 2) + ((v >> qsh) & 3)
            idx_ref[i >> 3, pl.ds((i & 7) * 16, 16)] = g
            return c
        lax.fori_loop(0, SLABS * 8, body, jnp.int32(0))

    def load_chunk(src_h, src_t, src_r, slab0, idx_h_v, idx_t_v, idx_r_v,
                   rows_h_v, rows_t_v, ent, sem):
        pltpu.sync_copy(src_h.at[pl.ds(slab0, SLABS)], idx_h_v)
        pltpu.sync_copy(src_t.at[pl.ds(slab0, SLABS)], idx_t_v)
        pltpu.sync_copy(src_r.at[pl.ds(slab0, SLABS)], idx_r_v)
        xform_ent(idx_h_v)
        xform_ent(idx_t_v)
        cps = []
        for j in range(SLABS):
            cps.append(pltpu.async_copy(
                ent.at[idx_h_v.at[j]], rows_h_v.at[pl.ds(j * 128, 128)], sem))
            cps.append(pltpu.async_copy(
                ent.at[idx_t_v.at[j]], rows_t_v.at[pl.ds(j * 128, 128)], sem))
        for c in cps:
            c.wait()

    @functools.partial(
        pl.kernel, mesh=mesh,
        out_type=jax.ShapeDtypeStruct((NW, 16), jnp.float32),
        compiler_params=pltpu.CompilerParams(
            needs_layout_passes=False, use_tc_tiling_on_sc=False),
        scratch_types=[
            pltpu.VMEM((n_rel, dim), jnp.float32),
            pltpu.VMEM((SLABS, 128), jnp.int32),
            pltpu.VMEM((SLABS, 128), jnp.int32),
            pltpu.VMEM((SLABS, 128), jnp.int32),
            pltpu.VMEM((P, 32), jnp.float32),
            pltpu.VMEM((P, 32), jnp.float32),
            pltpu.VMEM((P,), jnp.float32),
            pltpu.VMEM((16,), jnp.float32),
            pltpu.SemaphoreType.DMA,
        ])
    def sc_body(ph, pr, pt, nh, nr, nt, ent, rel, out,
                rel_v, idx_h_v, idx_t_v, idx_r_v, rows_h_v, rows_t_v,
                pos_s_v, acc_v, sem):
        wid = lax.axis_index("s") * NC + lax.axis_index("c")
        lanes = lax.iota(jnp.int32, 16)
        pltpu.sync_copy(rel, rel_v)

        load_chunk(ph, pt, pr, wid * SLABS, idx_h_v, idx_t_v, idx_r_v,
                   rows_h_v, rows_t_v, ent, sem)

        def pos_it(it, carry):
            sc = chunk_scores(rows_h_v, rows_t_v, rel_v, idx_r_v, it, lanes)
            pos_s_v[pl.ds(it * 16, 16)] = sc
            return carry
        lax.fori_loop(0, P // 16, pos_it, jnp.int32(0))

        iota4 = lanes >> 2
        acc = jnp.zeros((16,), jnp.float32)
        for n in range(CH):
            load_chunk(nh, nt, nr, (wid * CH + n) * SLABS,
                       idx_h_v, idx_t_v, idx_r_v, rows_h_v, rows_t_v, ent, sem)

            def neg_it(it, a, _n=n):
                sc = chunk_scores(rows_h_v, rows_t_v, rel_v, idx_r_v, it, lanes)
                pidx = iota4 + (_n * 128 + it * 4)
                ps = plsc.load_gather(pos_s_v, [pidx])
                return a + jnp.maximum(jnp.float32(margin) - ps + sc,
                                       jnp.float32(0.0))
            acc = lax.fori_loop(0, P // 16, neg_it, acc)

        acc_v[...] = acc
        pltpu.sync_copy(acc_v, out.at[wid])

    return sc_body


def _compact_body(x_ref, o_ref):
    dim, blk = x_ref.shape
    L = blk // 4
    stacked = jnp.concatenate(
        [x_ref[:, a * L:(a + 1) * L] for a in range(4)], axis=0)
    eye = jnp.eye(4 * dim, dtype=jnp.float32)
    dn = (((0,), (0,)), ((), ()))
    o_ref[...] = jax.lax.dot_general(stacked, eye, dn,
                                     preferred_element_type=jnp.float32)


def _compact_table(x, blk):
    n, dim = x.shape
    x_t = x.T
    grid = pl.cdiv(n, blk)
    out = pl.pallas_call(
        _compact_body,
        grid=(grid,),
        in_specs=[pl.BlockSpec((dim, blk), lambda i: (0, i))],
        out_specs=pl.BlockSpec((blk // 4, 4 * dim), lambda i: (i, 0)),
        out_shape=jax.ShapeDtypeStruct((grid * blk // 4, 4 * dim), jnp.float32),
    )(x_t)
    return out.reshape(grid * blk, dim)


def _reduce_loss(partials, neg_total):
    def body(p_ref, o_ref):
        o_ref[0, 0] = jnp.sum(p_ref[...]) * jnp.float32(1.0 / neg_total)

    out = pl.pallas_call(
        body,
        out_shape=jax.ShapeDtypeStruct((1, 1), jnp.float32),
        out_specs=pl.BlockSpec(memory_space=pltpu.SMEM),
    )(partials)
    return out[0, 0]


def kernel(pos_h, pos_r, pos_t, neg_h, neg_r, neg_t, ent_emb, rel_emb):
    B = pos_h.shape[0]
    NEG = neg_h.shape[0]
    n_rel, dim = rel_emb.shape
    sck = _make_sc_kernel(n_rel, dim, B, NEG, margin=1.0, ent_blk=65536)
    ph = pos_h.reshape(-1, 128)
    pr = pos_r.reshape(-1, 128)
    pt = pos_t.reshape(-1, 128)
    nh = neg_h.reshape(-1, 128)
    nr = neg_r.reshape(-1, 128)
    nt = neg_t.reshape(-1, 128)
    ent_c = _compact_table(ent_emb, blk=65536)
    partials = sck(ph, pr, pt, nh, nr, nt, ent_c, rel_emb)
    return _reduce_loss(partials, NEG)


# speedup vs baseline: 3.7445x; 1.0447x over previous
import functools

import jax
import jax.numpy as jnp
from jax import lax
from jax.experimental import pallas as pl
from jax.experimental.pallas import tpu as pltpu
from jax.experimental.pallas import tpu_sc as plsc


def _rsqrt16(s):
    s = jnp.maximum(s, jnp.float32(1e-24))
    i = plsc.bitcast(s, jnp.int32)
    i = jnp.int32(0x5F3759DF) - (i >> 1)
    y = plsc.bitcast(i, jnp.float32)
    for _ in range(3):
        y = y * (jnp.float32(1.5) - jnp.float32(0.5) * s * y * y)
    return y


def _make_sc_kernel(n_rel, dim, B, NEG, margin, ent_blk):
    info = plsc.get_sparse_core_info()
    NC, NS = info.num_cores, info.num_subcores
    NW = NC * NS
    P = B // NW
    CH = NEG // (NW * P)
    SLABS = P // 128
    assert dim == 32 and P % 128 == 0 and NEG == NW * P * CH and B == NW * P

    mesh = plsc.VectorSubcoreMesh(core_axis_name="c", subcore_axis_name="s")

    def chunk_scores(rows_h_v, rows_t_v, rel_v, idx_r_v, it, lanes):
        rvec = lanes + it * 16
        ridx = idx_r_v[it >> 3, pl.ds((it & 7) * 16, 16)]
        sh = jnp.zeros((16,), jnp.float32)
        st = jnp.zeros((16,), jnp.float32)
        for d in range(dim):
            dv = (lanes + d) & (dim - 1)
            hv = plsc.load_gather(rows_h_v, [rvec, dv])
            tv = plsc.load_gather(rows_t_v, [rvec, dv])
            sh = sh + hv * hv
            st = st + tv * tv
        ih = _rsqrt16(sh)
        iv = _rsqrt16(st)
        s = jnp.zeros((16,), jnp.float32)
        for d in range(dim):
            dv = (lanes + d) & (dim - 1)
            hv = plsc.load_gather(rows_h_v, [rvec, dv])
            tv = plsc.load_gather(rows_t_v, [rvec, dv])
            rv = plsc.load_gather(rel_v, [ridx, dv])
            s = s + jnp.abs(hv * ih + rv - tv * iv)
        return -s

    ql = ent_blk // 4
    qsh = ql.bit_length() - 1
    assert ent_blk & (ent_blk - 1) == 0 and (1 << qsh) == ql

    def xform_ent(idx_ref):
        def body(i, c):
            v = idx_ref[i >> 3, pl.ds((i & 7) * 16, 16)]
            g = (v & (-ent_blk)) + ((v & (ql - 1)) << 2) + ((v >> qsh) & 3)
            idx_ref[i >> 3, pl.ds((i & 7) * 16, 16)] = g
            return c
        lax.fori_loop(0, SLABS * 8, body, jnp.int32(0))

    def fire_chunk(src_h, src_t, src_r, slab0, buf, ent):
        idx_h_v, idx_t_v, idx_r_v, rows_h_v, rows_t_v, sem = buf
        pltpu.sync_copy(src_h.at[pl.ds(slab0, SLABS)], idx_h_v)
        pltpu.sync_copy(src_t.at[pl.ds(slab0, SLABS)], idx_t_v)
        pltpu.sync_copy(src_r.at[pl.ds(slab0, SLABS)], idx_r_v)
        xform_ent(idx_h_v)
        xform_ent(idx_t_v)
        cps = []
        for j in range(SLABS):
            cps.append(pltpu.async_copy(
                ent.at[idx_h_v.at[j]], rows_h_v.at[pl.ds(j * 128, 128)], sem))
            cps.append(pltpu.async_copy(
                ent.at[idx_t_v.at[j]], rows_t_v.at[pl.ds(j * 128, 128)], sem))
        return cps

    @functools.partial(
        pl.kernel, mesh=mesh,
        out_type=jax.ShapeDtypeStruct((NW, 16), jnp.float32),
        compiler_params=pltpu.CompilerParams(
            needs_layout_passes=False, use_tc_tiling_on_sc=False),
        scratch_types=[
            pltpu.VMEM((n_rel, dim), jnp.float32),
            pltpu.VMEM((SLABS, 128), jnp.int32),
            pltpu.VMEM((SLABS, 128), jnp.int32),
            pltpu.VMEM((SLABS, 128), jnp.int32),
            pltpu.VMEM((SLABS, 128), jnp.int32),
            pltpu.VMEM((SLABS, 128), jnp.int32),
            pltpu.VMEM((SLABS, 128), jnp.int32),
            pltpu.VMEM((P, 32), jnp.float32),
            pltpu.VMEM((P, 32), jnp.float32),
            pltpu.VMEM((P, 32), jnp.float32),
            pltpu.VMEM((P, 32), jnp.float32),
            pltpu.VMEM((P,), jnp.float32),
            pltpu.VMEM((16,), jnp.float32),
            pltpu.SemaphoreType.DMA,
            pltpu.SemaphoreType.DMA,
        ])
    def sc_body(ph, pr, pt, nh, nr, nt, ent, rel, out,
                rel_v, ih0, it0, ir0, ih1, it1, ir1, rh0, rt0, rh1, rt1,
                pos_s_v, acc_v, sem0, sem1):
        wid = lax.axis_index("s") * NC + lax.axis_index("c")
        lanes = lax.iota(jnp.int32, 16)
        pltpu.sync_copy(rel, rel_v)

        bufs = [(ih0, it0, ir0, rh0, rt0, sem0),
                (ih1, it1, ir1, rh1, rt1, sem1)]
        chunks = [(ph, pt, pr, wid * SLABS)] + [
            (nh, nt, nr, (wid * CH + n) * SLABS) for n in range(CH)]

        iota4 = lanes >> 2
        acc = jnp.zeros((16,), jnp.float32)
        cps = fire_chunk(*chunks[0], bufs[0], ent)
        for n in range(len(chunks)):
            _, _, idx_r_v, rows_h_v, rows_t_v, _ = bufs[n % 2]
            for c in cps:
                c.wait()
            if n + 1 < len(chunks):
                cps = fire_chunk(*chunks[n + 1], bufs[(n + 1) % 2], ent)
            if n == 0:
                def pos_it(it, carry):
                    sc = chunk_scores(rows_h_v, rows_t_v, rel_v, idx_r_v,
                                      it, lanes)
                    pos_s_v[pl.ds(it * 16, 16)] = sc
                    return carry
                lax.fori_loop(0, P // 16, pos_it, jnp.int32(0))
            else:
                def neg_it(it, a, _n=n - 1, _ir=idx_r_v, _rh=rows_h_v,
                           _rt=rows_t_v):
                    sc = chunk_scores(_rh, _rt, rel_v, _ir, it, lanes)
                    pidx = iota4 + (_n * 128 + it * 4)
                    ps = plsc.load_gather(pos_s_v, [pidx])
                    return a + jnp.maximum(jnp.float32(margin) - ps + sc,
                                           jnp.float32(0.0))
                acc = lax.fori_loop(0, P // 16, neg_it, acc)

        acc_v[...] = acc
        pltpu.sync_copy(acc_v, out.at[wid])

    return sc_body


def _compact_body(x_ref, o_ref):
    dim, blk = x_ref.shape
    L = blk // 4
    stacked = jnp.concatenate(
        [x_ref[:, a * L:(a + 1) * L] for a in range(4)], axis=0)
    eye = jnp.eye(4 * dim, dtype=jnp.float32)
    dn = (((0,), (0,)), ((), ()))
    o_ref[...] = jax.lax.dot_general(stacked, eye, dn,
                                     preferred_element_type=jnp.float32)


def _compact_table(x, blk):
    n, dim = x.shape
    x_t = x.T
    grid = pl.cdiv(n, blk)
    out = pl.pallas_call(
        _compact_body,
        grid=(grid,),
        in_specs=[pl.BlockSpec((dim, blk), lambda i: (0, i))],
        out_specs=pl.BlockSpec((blk // 4, 4 * dim), lambda i: (i, 0)),
        out_shape=jax.ShapeDtypeStruct((grid * blk // 4, 4 * dim), jnp.float32),
    )(x_t)
    return out.reshape(grid * blk, dim)


def _reduce_loss(partials, neg_total):
    def body(p_ref, o_ref):
        o_ref[0, 0] = jnp.sum(p_ref[...]) * jnp.float32(1.0 / neg_total)

    out = pl.pallas_call(
        body,
        out_shape=jax.ShapeDtypeStruct((1, 1), jnp.float32),
        out_specs=pl.BlockSpec(memory_space=pltpu.SMEM),
    )(partials)
    return out[0, 0]


def kernel(pos_h, pos_r, pos_t, neg_h, neg_r, neg_t, ent_emb, rel_emb):
    B = pos_h.shape[0]
    NEG = neg_h.shape[0]
    n_rel, dim = rel_emb.shape
    sck = _make_sc_kernel(n_rel, dim, B, NEG, margin=1.0, ent_blk=65536)
    ph = pos_h.reshape(-1, 128)
    pr = pos_r.reshape(-1, 128)
    pt = pos_t.reshape(-1, 128)
    nh = neg_h.reshape(-1, 128)
    nr = neg_r.reshape(-1, 128)
    nt = neg_t.reshape(-1, 128)
    ent_c = _compact_table(ent_emb, blk=65536)
    partials = sck(ph, pr, pt, nh, nr, nt, ent_c, rel_emb)
    return _reduce_loss(partials, NEG)
